# baseline (speedup 1.0000x reference)
import functools

import jax
import jax.numpy as jnp
from jax import lax
from jax.experimental import pallas as pl
from jax.experimental.pallas import tpu as pltpu
from jax.experimental.pallas import tpu_sc as plsc

BATCH = 16384
EMB = 64
PAIR = 2 * EMB
HID = 10

_info = plsc.get_sparse_core_info()
_NC, _NS = _info.num_cores, _info.num_subcores
NW = _NC * _NS
BPW = BATCH // NW
CHUNK = 128


def _sc_gather_body(uidx_hbm, midx_hbm, ut_hbm, mt_hbm, ue_hbm, me_hbm,
                    idx_v, rows_v, sem):
    wid = lax.axis_index("s") * _NC + lax.axis_index("c")
    base = wid * BPW
    for idx_hbm, t_hbm, o_hbm in ((uidx_hbm, ut_hbm, ue_hbm),
                                  (midx_hbm, mt_hbm, me_hbm)):
        pltpu.sync_copy(idx_hbm.at[pl.ds(base, BPW)], idx_v)
        handles = []
        for c in range(BPW // CHUNK):
            sl = pl.ds(c * CHUNK, CHUNK)
            handles.append(
                pltpu.async_copy(t_hbm.at[idx_v.at[sl]], rows_v.at[sl], sem))
        for h in handles:
            h.wait()
        pltpu.sync_copy(rows_v, o_hbm.at[pl.ds(base, BPW)])


_sc_gather = functools.partial(
    pl.kernel,
    out_type=[
        jax.ShapeDtypeStruct((BATCH, PAIR), jnp.float32),
        jax.ShapeDtypeStruct((BATCH, PAIR), jnp.float32),
    ],
    mesh=plsc.VectorSubcoreMesh(core_axis_name="c", subcore_axis_name="s"),
    scratch_types=[
        pltpu.VMEM((BPW,), jnp.int32),
        pltpu.VMEM((BPW, PAIR), jnp.float32),
        pltpu.SemaphoreType.DMA,
    ],
)(_sc_gather_body)


def _mlp_body(ue_ref, me_ref, uh_ref, mh_ref, w1u_ref, w1m_ref, b1_ref,
              w2_ref, b2_ref, out_ref):
    blk = ue_ref.shape[0]
    lane = lax.broadcasted_iota(jnp.int32, (blk, PAIR), 1)
    in_lo = (lane < EMB).astype(jnp.float32)
    usel = in_lo + uh_ref[...] * (1.0 - 2.0 * in_lo)
    msel = in_lo + mh_ref[...] * (1.0 - 2.0 * in_lo)
    ue = jnp.maximum(ue_ref[...], 0.0) * usel
    me = jnp.maximum(me_ref[...], 0.0) * msel
    h = (
        jnp.dot(ue, w1u_ref[...], preferred_element_type=jnp.float32)
        + jnp.dot(me, w1m_ref[...], preferred_element_type=jnp.float32)
        + b1_ref[...]
    )
    h = jnp.maximum(h, 0.0)
    out_ref[...] = jnp.dot(h, w2_ref[...], preferred_element_type=jnp.float32) + b2_ref[...]


def _mlp(ue, me, uh, mh, w1u2, w1m2, b1, w2, b2):
    blk = 2048
    grid = (BATCH // blk,)
    return pl.pallas_call(
        _mlp_body,
        grid=grid,
        in_specs=[
            pl.BlockSpec((blk, PAIR), lambda i: (i, 0)),
            pl.BlockSpec((blk, PAIR), lambda i: (i, 0)),
            pl.BlockSpec((blk, 1), lambda i: (i, 0)),
            pl.BlockSpec((blk, 1), lambda i: (i, 0)),
            pl.BlockSpec((PAIR, HID), lambda i: (0, 0)),
            pl.BlockSpec((PAIR, HID), lambda i: (0, 0)),
            pl.BlockSpec((1, HID), lambda i: (0, 0)),
            pl.BlockSpec((HID, 1), lambda i: (0, 0)),
            pl.BlockSpec((1, 1), lambda i: (0, 0)),
        ],
        out_specs=pl.BlockSpec((blk, 1), lambda i: (i, 0)),
        out_shape=jax.ShapeDtypeStruct((BATCH, 1), jnp.float32),
    )(ue, me, uh, mh, w1u2, w1m2, b1, w2, b2)


def kernel(user, movie, user_table, movie_table, W1, b1, W2, b2):
    user = user.astype(jnp.int32)
    movie = movie.astype(jnp.int32)
    ut2 = user_table.reshape(user_table.shape[0] // 2, PAIR)
    mt2 = movie_table.reshape(movie_table.shape[0] // 2, PAIR)
    uidx = user >> 1
    midx = movie >> 1
    uh = (user & 1).astype(jnp.float32).reshape(BATCH, 1)
    mh = (movie & 1).astype(jnp.float32).reshape(BATCH, 1)
    ue, me = _sc_gather(uidx, midx, ut2, mt2)
    w1u2 = jnp.concatenate([W1[:, :EMB].T, W1[:, :EMB].T], axis=0)
    w1m2 = jnp.concatenate([W1[:, EMB:].T, W1[:, EMB:].T], axis=0)
    return _mlp(ue, me, uh, mh, w1u2, w1m2,
                b1.reshape(1, HID), W2.T, b2.reshape(1, 1))


# speedup vs baseline: 2.2851x; 2.2851x over previous
import functools

import jax
import jax.numpy as jnp
from jax import lax
from jax.experimental import pallas as pl
from jax.experimental.pallas import tpu as pltpu
from jax.experimental.pallas import tpu_sc as plsc

BATCH = 16384
EMB = 64
PAD = 2 * EMB
HID = 10
NUSER = 1000000
NMOVIE = 100000

_info = plsc.get_sparse_core_info()
_NC, _NS = _info.num_cores, _info.num_subcores
NW = _NC * _NS

BLK_R = 1024
NBLK_U = 976
BOUND_U = NBLK_U * BLK_R
NBLK_M = 97
BOUND_M = NBLK_M * BLK_R
FLUSH = 64
STG = BATCH + NW
SENT = 0x7FFFFFF0
TRASH = BATCH + 16
IDXC = 4096


def _sweep_table(tT_hbm, idx_hbm, o_hbm, b0, nb_full, lo, hi,
                 allidx_v, cand_r, cand_p, hit_h, blk_v, flush_v,
                 flush_p, sem, wid):
    iota = lax.iota(jnp.int32, 16)

    n_c = jnp.int32(0)
    for sc in range(BATCH // IDXC):
        pltpu.sync_copy(idx_hbm.at[pl.ds(sc * IDXC, IDXC)], allidx_v)

        def cscan(ch, n_c, sc=sc):
            chunk = allidx_v[pl.ds(ch * 16, 16)]
            m = (chunk >= lo) & (chunk < hi)
            pref = plsc.cumsum(m.astype(jnp.int32))
            dst = jnp.where(m, n_c + pref - 1, TRASH)
            plsc.store_scatter(cand_r, [dst], chunk)
            plsc.store_scatter(cand_p, [dst], (sc * IDXC + ch * 16) + iota)
            return n_c + pref[15]

        n_c = lax.fori_loop(0, IDXC // 16, cscan, n_c)
    cand_r[pl.ds(n_c, 16)] = jnp.full((16,), SENT, jnp.int32)
    cand_p[pl.ds(n_c, 16)] = jnp.full((16,), -1, jnp.int32)
    n_cg = (n_c + 15) // 16

    def process_block(r0, width):
        def hscan(g, n_h):
            cr = cand_r[pl.ds(g * 16, 16)]
            cp = cand_p[pl.ds(g * 16, 16)]
            m = (cr >= r0) & (cr < r0 + width)
            pref = plsc.cumsum(m.astype(jnp.int32))
            dst = jnp.where(m, n_h + pref - 1, TRASH)
            plsc.store_scatter(hit_h, [dst], cp * BLK_R + (cr - r0))
            return n_h + pref[15]

        n_h = lax.fori_loop(0, n_cg, hscan, jnp.int32(0))
        hit_h[pl.ds(n_h, 16)] = jnp.full((16,), -1, jnp.int32)

        def hproc(g, carry):
            h16 = hit_h[pl.ds(g * 16, 16)]
            hr = h16 & (BLK_R - 1)
            hp = h16 // BLK_R
            flush_p[0, pl.ds(0, 16)] = jnp.where(h16 >= 0, hp, BATCH + wid)
            for l in range(16):
                rloc = hr[l]
                col = jnp.full((16,), rloc, jnp.int32)
                for k in range(4):
                    vals = plsc.load_gather(blk_v, [iota + 16 * k, col])
                    flush_v[l, pl.ds(16 * k, 16)] = vals
            pltpu.async_copy(flush_v, o_hbm.at[flush_p.at[0]], sem).wait()
            return carry

        lax.fori_loop(0, (n_h + 15) // 16, hproc, jnp.int32(0))

    def sweep(b, carry):
        r0 = b * BLK_R
        pltpu.sync_copy(tT_hbm.at[:, pl.ds(r0, BLK_R)], blk_v)
        process_block(r0, BLK_R)
        return carry

    lax.fori_loop(b0, b0 + nb_full, sweep, jnp.int32(0))


def _sc_gather_body(user_hbm, movie_hbm, utT_hbm, mtT_hbm, ue_hbm, me_hbm,
                    allidx_v, cand_r, cand_p, hit_h, blk_v, flush_v,
                    flush_p, sem):
    wid = lax.axis_index("s") * _NC + lax.axis_index("c")

    for s in range(16):
        for k in range(4):
            flush_v[s, pl.ds(EMB + 16 * k, 16)] = jnp.zeros((16,), jnp.float32)

    b0u = jnp.where(wid < 16, wid * 31, wid * 30 + 16)
    nbu = jnp.where(wid < 16, 31, 30)
    _sweep_table(utT_hbm, user_hbm, ue_hbm, b0u, nbu,
                 b0u * BLK_R, (b0u + nbu) * BLK_R,
                 allidx_v, cand_r, cand_p, hit_h, blk_v, flush_v,
                 flush_p, sem, wid)

    b0m = wid * 3 + jnp.minimum(wid, 1)
    nbm = jnp.where(wid == 0, 4, 3)
    _sweep_table(mtT_hbm, movie_hbm, me_hbm, b0m, nbm,
                 b0m * BLK_R, (b0m + nbm) * BLK_R,
                 allidx_v, cand_r, cand_p, hit_h, blk_v, flush_v,
                 flush_p, sem, wid)


_sc_gather = functools.partial(
    pl.kernel,
    out_type=[
        jax.ShapeDtypeStruct((STG, PAD), jnp.float32),
        jax.ShapeDtypeStruct((STG, PAD), jnp.float32),
    ],
    mesh=plsc.VectorSubcoreMesh(core_axis_name="c", subcore_axis_name="s"),
    scratch_types=[
        pltpu.VMEM((IDXC,), jnp.int32),
        pltpu.VMEM((BATCH + 32,), jnp.int32),
        pltpu.VMEM((BATCH + 32,), jnp.int32),
        pltpu.VMEM((BATCH + 32,), jnp.int32),
        pltpu.VMEM((EMB, BLK_R), jnp.float32),
        pltpu.VMEM((16, PAD), jnp.float32),
        pltpu.VMEM((1, 16), jnp.int32),
        pltpu.SemaphoreType.DMA,
    ],
    compiler_params=pltpu.CompilerParams(needs_layout_passes=False),
)(_sc_gather_body)


def _mlp_body(ue_ref, me_ref, w1u_ref, w1m_ref, b1_ref, w2_ref, b2_ref,
              out_ref):
    ue = jnp.maximum(ue_ref[...], 0.0)
    me = jnp.maximum(me_ref[...], 0.0)
    h = (
        jnp.dot(ue, w1u_ref[...], preferred_element_type=jnp.float32)
        + jnp.dot(me, w1m_ref[...], preferred_element_type=jnp.float32)
        + b1_ref[...]
    )
    h = jnp.maximum(h, 0.0)
    out_ref[...] = (
        jnp.dot(h, w2_ref[...], preferred_element_type=jnp.float32)
        + b2_ref[...]
    )


def _mlp(ue, me, w1u, w1m, b1, w2, b2):
    blk = 2048
    grid = (BATCH // blk,)
    return pl.pallas_call(
        _mlp_body,
        grid=grid,
        in_specs=[
            pl.BlockSpec((blk, PAD), lambda i: (i, 0)),
            pl.BlockSpec((blk, PAD), lambda i: (i, 0)),
            pl.BlockSpec((PAD, HID), lambda i: (0, 0)),
            pl.BlockSpec((PAD, HID), lambda i: (0, 0)),
            pl.BlockSpec((1, HID), lambda i: (0, 0)),
            pl.BlockSpec((HID, 1), lambda i: (0, 0)),
            pl.BlockSpec((1, 1), lambda i: (0, 0)),
        ],
        out_specs=pl.BlockSpec((blk, 1), lambda i: (i, 0)),
        out_shape=jax.ShapeDtypeStruct((BATCH, 1), jnp.float32),
    )(ue, me, w1u, w1m, b1, w2, b2)


def kernel(user, movie, user_table, movie_table, W1, b1, W2, b2):
    user = user.astype(jnp.int32)
    movie = movie.astype(jnp.int32)
    utT = user_table.T
    mtT = movie_table.T
    ue_st, me_st = _sc_gather(user, movie, utT, mtT)
    ue_st = ue_st[:BATCH]
    me_st = me_st[:BATCH]

    tail_u = user_table[BOUND_U:]
    tail_m = movie_table[BOUND_M:]
    mu = user >= BOUND_U
    mm = movie >= BOUND_M
    pu = jnp.pad(jnp.take(tail_u, jnp.clip(user - BOUND_U, 0, NUSER - BOUND_U - 1), axis=0),
                 ((0, 0), (0, PAD - EMB)))
    pm = jnp.pad(jnp.take(tail_m, jnp.clip(movie - BOUND_M, 0, NMOVIE - BOUND_M - 1), axis=0),
                 ((0, 0), (0, PAD - EMB)))
    ue_st = jnp.where(mu[:, None], pu, ue_st)
    me_st = jnp.where(mm[:, None], pm, me_st)

    z = jnp.zeros((EMB, HID), jnp.float32)
    w1u = jnp.concatenate([W1[:, :EMB].T, z], axis=0)
    w1m = jnp.concatenate([W1[:, EMB:].T, z], axis=0)
    return _mlp(ue_st, me_st, w1u, w1m,
                b1.reshape(1, HID), W2.T, b2.reshape(1, 1))
